# baseline (speedup 1.0000x reference)
import functools

import numpy as np

import jax
import jax.numpy as jnp
from jax.experimental import pallas as pl
from jax.experimental.pallas import tpu as pltpu

_IDX_MID = (2, 0, 0, 1, 1, 3, 3, 5, 4, 5, 4, 7, 6, 7, 6, 9, 8, 11, 11, 9,
            10, 10, 8, 12, 12, 13, 13, 14, 14, 15, 15, 16, 17, 17, 18, 18,
            19, 19, 20, 20, 21, 21, 22, 22, 23, 23, 24, 24, 25, 25, 26, 26,
            27, 27, 28, 28, 29, 30, 30, 31, 32, 32, 31, 33, 33, 34, 35, 35,
            34, 36, 36, 37, 37, 38, 29, 38, 39, 39, 40, 40, 16, 41, 41, 42,
            42, 43, 43, 44, 44, 45, 45, 46, 47, 47, 48, 48, 49, 49, 50, 50,
            51, 51, 52, 52, 53, 53, 54, 54, 55, 55, 56, 56, 57, 58, 58, 59,
            59, 60, 60, 61, 61, 62, 62, 63, 63, 64, 64, 65, 65, 66, 66, 67,
            67)

_IDX_LOW = (0, 0, 1, 1, 2, 2, 3, 3, 4, 4, 5, 5, 6, 7, 8, 9, 10, 9, 8, 7,
            6, 11, 12, 13, 12, 11, 13, 14, 15, 14, 15, 16, 17, 18, 16, 17,
            18, 19, 10, 19, 20, 20, 21, 21, 22, 22, 23, 24, 25, 26, 27, 28,
            29, 30, 31, 32, 33, 23, 24, 25, 26, 27, 28, 29, 30, 31, 32, 33)


@functools.lru_cache(maxsize=None)
def _make_unpool(batch, j_in, d, idx):
    j_out = len(idx)

    w = 6
    steps = -(-j_out // w)
    src_np = np.zeros((w, steps), np.int32)
    for t in range(w * steps):
        src_np[t % w, t // w] = idx[min(t, j_out - 1)]

    def body(src_ref, *refs):
        o_ref = refs[w]
        for k in range(w):
            o_ref[k] = refs[k][0]

    def in_map(k):
        return lambda u, src: (src[k, u], 0, 0)

    grid_spec = pltpu.PrefetchScalarGridSpec(
        num_scalar_prefetch=1,
        grid=(steps,),
        in_specs=[pl.BlockSpec((1, d, batch), in_map(k)) for k in range(w)],
        out_specs=pl.BlockSpec((w, d, batch), lambda u, src: (u, 0, 0)),
    )
    f = pl.pallas_call(
        body,
        grid_spec=grid_spec,
        out_shape=jax.ShapeDtypeStruct((j_out, d, batch), jnp.float32),
    )

    def run(x_t):
        return f(jnp.asarray(src_np), *([x_t] * w))

    return run


def kernel(x):
    batch, j_in, d = x.shape
    idx = _IDX_MID if j_in == 68 else _IDX_LOW
    run = _make_unpool(batch, j_in, d, idx)
    x_t = jnp.transpose(x, (1, 2, 0))
    out_t = run(x_t)
    return jnp.transpose(out_t, (2, 0, 1))


# speedup vs baseline: 1.0370x; 1.0370x over previous
import functools

import numpy as np

import jax
import jax.numpy as jnp
from jax.experimental import pallas as pl
from jax.experimental.pallas import tpu as pltpu

_IDX_MID = (2, 0, 0, 1, 1, 3, 3, 5, 4, 5, 4, 7, 6, 7, 6, 9, 8, 11, 11, 9,
            10, 10, 8, 12, 12, 13, 13, 14, 14, 15, 15, 16, 17, 17, 18, 18,
            19, 19, 20, 20, 21, 21, 22, 22, 23, 23, 24, 24, 25, 25, 26, 26,
            27, 27, 28, 28, 29, 30, 30, 31, 32, 32, 31, 33, 33, 34, 35, 35,
            34, 36, 36, 37, 37, 38, 29, 38, 39, 39, 40, 40, 16, 41, 41, 42,
            42, 43, 43, 44, 44, 45, 45, 46, 47, 47, 48, 48, 49, 49, 50, 50,
            51, 51, 52, 52, 53, 53, 54, 54, 55, 55, 56, 56, 57, 58, 58, 59,
            59, 60, 60, 61, 61, 62, 62, 63, 63, 64, 64, 65, 65, 66, 66, 67,
            67)

_IDX_LOW = (0, 0, 1, 1, 2, 2, 3, 3, 4, 4, 5, 5, 6, 7, 8, 9, 10, 9, 8, 7,
            6, 11, 12, 13, 12, 11, 13, 14, 15, 14, 15, 16, 17, 18, 16, 17,
            18, 19, 10, 19, 20, 20, 21, 21, 22, 22, 23, 24, 25, 26, 27, 28,
            29, 30, 31, 32, 33, 23, 24, 25, 26, 27, 28, 29, 30, 31, 32, 33)


@functools.lru_cache(maxsize=None)
def _make_unpool(batch, j_in, d, idx):
    j_out = len(idx)

    w = 7
    steps = -(-j_out // w)
    src_np = np.zeros((w, steps), np.int32)
    for t in range(w * steps):
        src_np[t % w, t // w] = idx[min(t, j_out - 1)]

    def body(src_ref, *refs):
        o_ref = refs[w]
        for k in range(w):
            o_ref[k] = refs[k][0]

    def in_map(k):
        return lambda u, src: (src[k, u], 0, 0)

    grid_spec = pltpu.PrefetchScalarGridSpec(
        num_scalar_prefetch=1,
        grid=(steps,),
        in_specs=[pl.BlockSpec((1, d, batch), in_map(k)) for k in range(w)],
        out_specs=pl.BlockSpec((w, d, batch), lambda u, src: (u, 0, 0)),
    )
    f = pl.pallas_call(
        body,
        grid_spec=grid_spec,
        out_shape=jax.ShapeDtypeStruct((j_out, d, batch), jnp.float32),
    )

    def run(x_t):
        return f(jnp.asarray(src_np), *([x_t] * w))

    return run


def kernel(x):
    batch, j_in, d = x.shape
    idx = _IDX_MID if j_in == 68 else _IDX_LOW
    run = _make_unpool(batch, j_in, d, idx)
    x_t = jnp.transpose(x, (1, 2, 0))
    out_t = run(x_t)
    return jnp.transpose(out_t, (2, 0, 1))
